# baseline (speedup 1.0000x reference)
import functools

import jax
import jax.numpy as jnp
from jax import lax
from jax.experimental import pallas as pl
from jax.experimental.pallas import tpu as pltpu
from jax.experimental.pallas import tpu_sc as plsc

N = 10000
E = 320000
D = 128
NC = 2
NS = 16
NW = NC * NS
NPT = 320
NPAD = NW * NPT
EPT = E // NW
GC = 80
SCC = 128
DC = 2000
BKCAP = 10112
BLK = 128
SHIFT = 20
MASK = (1 << SHIFT) - 1
MAGIC = 52429
QE = 452608

_mesh = plsc.VectorSubcoreMesh(core_axis_name="c", subcore_axis_name="s")


def _wid():
    return lax.axis_index("s") * NC + lax.axis_index("c")


def _al8(v):
    return pl.multiple_of(v, 8)


@functools.partial(
    pl.kernel,
    out_type=(
        jax.ShapeDtypeStruct((NW * NW * BKCAP,), jnp.int32),
        jax.ShapeDtypeStruct((NW * NW * BKCAP,), jnp.int32),
        jax.ShapeDtypeStruct((NW * NW * 16,), jnp.int32),
    ),
    scratch_types=[
        pltpu.VMEM((DC,), jnp.int32),
        pltpu.VMEM((DC,), jnp.int32),
        pltpu.VMEM((NW * BLK,), jnp.int32),
        pltpu.VMEM((NW * BLK,), jnp.int32),
        pltpu.VMEM((NW * 16,), jnp.int32),
        pltpu.VMEM((NW * 16,), jnp.int32),
        pltpu.VMEM((NW * 16,), jnp.int32),
    ],
    mesh=_mesh,
)
def _bin_edges(dst_hbm, src_hbm, bins_hbm, bsrc_hbm, counts_hbm,
               dchunk, schunk, bblk, sblk, bcb, fbb, cnt):
    t = _wid()
    iota = lax.iota(jnp.int32, 16)
    zero = jnp.zeros((16,), jnp.int32)

    def zinit(k, _):
        bcb[pl.ds(k * 16, 16)] = zero
        fbb[pl.ds(k * 16, 16)] = zero
        return 0

    lax.fori_loop(0, NW, zinit, 0)

    def chunk_body(ch, _):
        ebase = t * EPT + ch * DC
        pltpu.sync_copy(dst_hbm.at[pl.ds(_al8(ebase), DC)], dchunk)
        pltpu.sync_copy(src_hbm.at[pl.ds(_al8(ebase), DC)], schunk)

        def vec_body(j, _):
            dvec = dchunk[pl.ds(j * 16, 16)]
            svec = schunk[pl.ds(j * 16, 16)]
            for i in range(16):
                d = dvec[i]
                s = svec[i]
                bkt = (d * MAGIC) >> 24
                val = ((d - bkt * NPT) << SHIFT) | (ebase + j * 16 + i)
                bcw = bcb[pl.ds(bkt * 16, 16)]
                bc = bcw[0]
                wpos = bkt * BLK + (bc & ~15)
                lane = iota == (bc & 15)
                w = bblk[pl.ds(wpos, 16)]
                bblk[pl.ds(wpos, 16)] = jnp.where(lane, val, w)
                w2 = sblk[pl.ds(wpos, 16)]
                sblk[pl.ds(wpos, 16)] = jnp.where(lane, s, w2)
                bcb[pl.ds(bkt * 16, 16)] = bcw + 1

                @pl.when(bc + 1 == BLK)
                def _():
                    fbw = fbb[pl.ds(bkt * 16, 16)]
                    dst_off = (t * NW + bkt) * BKCAP + fbw[0] * BLK
                    pltpu.sync_copy(
                        bblk.at[pl.ds(bkt * BLK, BLK)],
                        bins_hbm.at[pl.ds(_al8(dst_off), BLK)])
                    pltpu.sync_copy(
                        sblk.at[pl.ds(bkt * BLK, BLK)],
                        bsrc_hbm.at[pl.ds(_al8(dst_off), BLK)])
                    fbb[pl.ds(bkt * 16, 16)] = fbw + 1
                    bcb[pl.ds(bkt * 16, 16)] = zero
            return 0

        lax.fori_loop(0, DC // 16, vec_body, 0)
        return 0

    lax.fori_loop(0, EPT // DC, chunk_body, 0)

    for k in range(NW):
        bc = bcb[pl.ds(k * 16, 16)][0]
        fb = fbb[pl.ds(k * 16, 16)][0]
        dst_off = (t * NW + k) * BKCAP + fb * BLK
        pltpu.sync_copy(bblk.at[pl.ds(k * BLK, BLK)],
                        bins_hbm.at[pl.ds(_al8(dst_off), BLK)])
        pltpu.sync_copy(sblk.at[pl.ds(k * BLK, BLK)],
                        bsrc_hbm.at[pl.ds(_al8(dst_off), BLK)])
        cnt[pl.ds(k * 16, 16)] = jnp.full((16,), fb * BLK + bc, jnp.int32)
    pltpu.sync_copy(cnt, counts_hbm.at[pl.ds(_al8(t * NW * 16), NW * 16)])


@functools.partial(
    pl.kernel,
    out_type=jax.ShapeDtypeStruct((QE, D), jnp.float32),
    scratch_types=[
        pltpu.VMEM((NW * NW * 16,), jnp.int32),
        pltpu.VMEM((NW * 16,), jnp.int32),
        pltpu.VMEM((SCC,), jnp.int32),
        pltpu.VMEM((SCC,), jnp.int32),
        pltpu.VMEM((SCC,), jnp.int32),
        pltpu.VMEM((SCC,), jnp.int32),
        pltpu.VMEM((SCC, D), jnp.float32),
        pltpu.VMEM((SCC, D), jnp.float32),
        pltpu.VMEM((SCC, D), jnp.float32),
        pltpu.VMEM((SCC, D), jnp.float32),
        pltpu.VMEM((SCC, D), jnp.float32),
        pltpu.VMEM((SCC, D), jnp.float32),
        pltpu.SemaphoreType.DMA,
        pltpu.SemaphoreType.DMA,
        pltpu.SemaphoreType.DMA,
        pltpu.SemaphoreType.DMA,
    ],
    mesh=_mesh,
)
def _gather_q(u_hbm, v_hbm, bins_hbm, bsrc_hbm, counts_hbm, q_hbm,
              cnts, qbv, ixd0, ixs0, ixd1, ixs1,
              ub0, vb0, ub1, vb1, qb0, qb1, sg0, sg1, so0, so1):
    k = _wid()
    iota = lax.iota(jnp.int32, 16)
    pltpu.sync_copy(counts_hbm, cnts)

    def accum(kp, acc):
        def accum_t(t, a):
            c = cnts[pl.ds((t * NW + kp) * 16, 16)][0]
            return a + ((c + 127) & ~127)
        return lax.fori_loop(0, NW, accum_t, acc)

    base = lax.fori_loop(0, k, accum, jnp.int32(0))

    def setb(t, b):
        qbv[pl.ds(t * 16, 16)] = jnp.full((16,), b, jnp.int32)
        c = cnts[pl.ds((t * NW + k) * 16, 16)][0]
        return b + ((c + 127) & ~127)

    lax.fori_loop(0, NW, setb, base)

    def seg_body(t, _):
        ct = cnts[pl.ds((t * NW + k) * 16, 16)][0]
        qbase = qbv[pl.ds(t * 16, 16)][0]
        nch = (ct + (SCC - 1)) >> 7
        sbase = (t * NW + k) * BKCAP

        def load_chunk(g, ixd, ixs, ub, vb, sg):
            pltpu.sync_copy(bins_hbm.at[pl.ds(_al8(sbase + g * SCC), SCC)], ixd)
            pltpu.sync_copy(bsrc_hbm.at[pl.ds(_al8(sbase + g * SCC), SCC)], ixs)

            def idx_body(j, _):
                sl = pl.ds(j * 16, 16)
                valid = (g * SCC + j * 16 + iota) < ct
                dl = lax.shift_right_logical(ixd[sl], SHIFT)
                ixd[sl] = jnp.where(valid, k * NPT + dl, 0)
                ixs[sl] = jnp.where(valid, ixs[sl], 0)
                return 0

            lax.fori_loop(0, SCC // 16, idx_body, 0)
            pltpu.async_copy(u_hbm.at[ixd], ub, sg)
            pltpu.async_copy(v_hbm.at[ixs], vb, sg)

        @pl.when(nch > 0)
        def _():
            load_chunk(0, ixd0, ixs0, ub0, vb0, sg0)

        def compute_chunk(g, ixd, ub, vb, sg, qb, so):
            pltpu.make_async_copy(u_hbm.at[ixd], ub, sg).wait()
            pltpu.make_async_copy(u_hbm.at[ixd], vb, sg).wait()

            @pl.when(g >= 2)
            def _():
                pltpu.make_async_copy(qb, q_hbm.at[pl.ds(0, SCC)], so).wait()

            def row(r, _):
                for kk in range(D // 16):
                    c = pl.ds(kk * 16, 16)
                    qb[r, c] = jnp.maximum(ub[r, c] + vb[r, c], 0.0)
                return 0

            lax.fori_loop(0, SCC, row, 0)
            pltpu.async_copy(qb, q_hbm.at[pl.ds(_al8(qbase + g * SCC), SCC)],
                             so)

        def halfstep(g, ixd, ixs, ub, vb, sg, qb, so,
                     ixdn, ixsn, ubn, vbn, sgn):
            @pl.when(g < nch)
            def _():
                @pl.when(g + 1 < nch)
                def _():
                    load_chunk(g + 1, ixdn, ixsn, ubn, vbn, sgn)

                compute_chunk(g, ixd, ub, vb, sg, qb, so)

        def pair(pp, _):
            g = pp * 2
            halfstep(g, ixd0, ixs0, ub0, vb0, sg0, qb0, so0,
                     ixd1, ixs1, ub1, vb1, sg1)
            halfstep(g + 1, ixd1, ixs1, ub1, vb1, sg1, qb1, so1,
                     ixd0, ixs0, ub0, vb0, sg0)
            return 0

        lax.fori_loop(0, (nch + 1) >> 1, pair, 0)
        @pl.when(nch >= 2)
        def _():
            pltpu.make_async_copy(qb0, q_hbm.at[pl.ds(0, SCC)], so0).wait()
            pltpu.make_async_copy(qb1, q_hbm.at[pl.ds(0, SCC)], so1).wait()

        @pl.when(nch == 1)
        def _():
            pltpu.make_async_copy(qb0, q_hbm.at[pl.ds(0, SCC)], so0).wait()
        return 0

    lax.fori_loop(0, NW, seg_body, 0)


@functools.partial(
    pl.kernel,
    out_type=jax.ShapeDtypeStruct((8 * NPAD * 16,), jnp.float32),
    scratch_types=[
        pltpu.VMEM((NW * NW * 16,), jnp.int32),
        pltpu.VMEM((NW * 16,), jnp.int32),
        pltpu.VMEM((SCC,), jnp.int32),
        pltpu.VMEM((SCC,), jnp.int32),
        pltpu.VMEM((SCC, D), jnp.float32),
        pltpu.VMEM((SCC, D), jnp.float32),
    ] + [pltpu.VMEM(((NPT + 1) * 16,), jnp.float32) for _ in range(8)] + [
        pltpu.SemaphoreType.DMA,
        pltpu.SemaphoreType.DMA,
    ],
    mesh=_mesh,
)
def _scatter_max(z_hbm, bins_hbm, counts_hbm, agg_hbm,
                 cnts, qbv, pk0, pk1, zb0, zb1,
                 s0, s1, s2, s3, s4, s5, s6, s7, sz0, sz1):
    k = _wid()
    iota = lax.iota(jnp.int32, 16)
    zero = jnp.zeros((16,), jnp.float32)
    slabs = (s0, s1, s2, s3, s4, s5, s6, s7)

    def init_row(r, _):
        for kk in range(8):
            slabs[kk][pl.ds(r * 16, 16)] = zero
        return 0

    lax.fori_loop(0, NPT + 1, init_row, 0)

    pltpu.sync_copy(counts_hbm, cnts)

    def accum(kp, acc):
        def accum_t(t, a):
            c = cnts[pl.ds((t * NW + kp) * 16, 16)][0]
            return a + ((c + 127) & ~127)
        return lax.fori_loop(0, NW, accum_t, acc)

    base = lax.fori_loop(0, k, accum, jnp.int32(0))

    def setb(t, b):
        qbv[pl.ds(t * 16, 16)] = jnp.full((16,), b, jnp.int32)
        c = cnts[pl.ds((t * NW + k) * 16, 16)][0]
        return b + ((c + 127) & ~127)

    lax.fori_loop(0, NW, setb, base)

    def seg_body(t, _):
        ct = cnts[pl.ds((t * NW + k) * 16, 16)][0]
        qbase = qbv[pl.ds(t * 16, 16)][0]
        nch = (ct + (SCC - 1)) >> 7
        sbase = (t * NW + k) * BKCAP

        def load_chunk(g, pk, zb, sz):
            pltpu.sync_copy(bins_hbm.at[pl.ds(_al8(sbase + g * SCC), SCC)], pk)
            pltpu.async_copy(z_hbm.at[pl.ds(_al8(qbase + g * SCC), SCC)],
                             zb, sz)

        @pl.when(nch > 0)
        def _():
            load_chunk(0, pk0, zb0, sz0)

        def reduce_chunk(g, pk, zb, sz):
            pltpu.make_async_copy(z_hbm.at[pl.ds(0, SCC)], zb, sz).wait()

            def grp_body(j, _):
                p = pk[pl.ds(j * 16, 16)]
                valid = (g * SCC + j * 16 + iota) < ct
                dlv = jnp.where(valid, lax.shift_right_logical(p, SHIFT), NPT)
                for i in range(16):
                    r16 = dlv[i] * 16
                    e = j * 16 + i
                    for kk in range(8):
                        zv = zb[e, pl.ds(kk * 16, 16)]
                        av = slabs[kk][pl.ds(r16, 16)]
                        slabs[kk][pl.ds(r16, 16)] = jnp.maximum(av, zv)
                return 0

            lax.fori_loop(0, SCC // 16, grp_body, 0)

        def halfstep(g, pk, zb, sz, pkn, zbn, szn):
            @pl.when(g < nch)
            def _():
                @pl.when(g + 1 < nch)
                def _():
                    load_chunk(g + 1, pkn, zbn, szn)

                reduce_chunk(g, pk, zb, sz)

        def pair(pp, _):
            g = pp * 2
            halfstep(g, pk0, zb0, sz0, pk1, zb1, sz1)
            halfstep(g + 1, pk1, zb1, sz1, pk0, zb0, sz0)
            return 0

        lax.fori_loop(0, (nch + 1) >> 1, pair, 0)
        return 0

    lax.fori_loop(0, NW, seg_body, 0)
    for kk in range(8):
        pltpu.sync_copy(
            slabs[kk].at[pl.ds(0, NPT * 16)],
            agg_hbm.at[pl.ds(_al8(kk * NPAD * 16 + k * NPT * 16), NPT * 16)])


_RB = 1000
_EB = 2048


def _proj_body(x_ref, w_ref, b_ref, o_ref):
    o_ref[...] = jnp.dot(x_ref[...], w_ref[...],
                         preferred_element_type=jnp.float32) + b_ref[...]


def _proj(x, Wp, bp2):
    return pl.pallas_call(
        _proj_body,
        grid=(N // _RB,),
        in_specs=[
            pl.BlockSpec((_RB, D), lambda i: (i, 0)),
            pl.BlockSpec((D, D), lambda i: (0, 0)),
            pl.BlockSpec((1, D), lambda i: (0, 0)),
        ],
        out_specs=pl.BlockSpec((_RB, D), lambda i: (i, 0)),
        out_shape=jax.ShapeDtypeStruct((N, D), jnp.float32),
    )(x, Wp, bp2)


def _uv0_body(h_ref, wd_ref, b1_ref, wb_ref, u_ref, v_ref):
    h = h_ref[...]
    u_ref[...] = jnp.dot(h, wd_ref[...],
                         preferred_element_type=jnp.float32) + b1_ref[...]
    v_ref[...] = jnp.dot(h, wb_ref[...], preferred_element_type=jnp.float32)


def _uv0(h, Wd, b12, Wb):
    return pl.pallas_call(
        _uv0_body,
        grid=(N // _RB,),
        in_specs=[
            pl.BlockSpec((_RB, D), lambda i: (i, 0)),
            pl.BlockSpec((D, D), lambda i: (0, 0)),
            pl.BlockSpec((1, D), lambda i: (0, 0)),
            pl.BlockSpec((D, D), lambda i: (0, 0)),
        ],
        out_specs=[
            pl.BlockSpec((_RB, D), lambda i: (i, 0)),
            pl.BlockSpec((_RB, D), lambda i: (i, 0)),
        ],
        out_shape=[
            jax.ShapeDtypeStruct((N, D), jnp.float32),
            jax.ShapeDtypeStruct((N, D), jnp.float32),
        ],
    )(h, Wd, b12, Wb)


def _uvres_body(a_ref, h_ref, wd_ref, b1_ref, wb_ref, hn_ref, u_ref, v_ref):
    hn = a_ref[...] + h_ref[...]
    hn_ref[...] = hn
    u_ref[...] = jnp.dot(hn, wd_ref[...],
                         preferred_element_type=jnp.float32) + b1_ref[...]
    v_ref[...] = jnp.dot(hn, wb_ref[...], preferred_element_type=jnp.float32)


def _uvres(a, h, Wd, b12, Wb):
    return pl.pallas_call(
        _uvres_body,
        grid=(N // _RB,),
        in_specs=[
            pl.BlockSpec((_RB, D), lambda i: (i, 0)),
            pl.BlockSpec((_RB, D), lambda i: (i, 0)),
            pl.BlockSpec((D, D), lambda i: (0, 0)),
            pl.BlockSpec((1, D), lambda i: (0, 0)),
            pl.BlockSpec((D, D), lambda i: (0, 0)),
        ],
        out_specs=[
            pl.BlockSpec((_RB, D), lambda i: (i, 0)),
            pl.BlockSpec((_RB, D), lambda i: (i, 0)),
            pl.BlockSpec((_RB, D), lambda i: (i, 0)),
        ],
        out_shape=[
            jax.ShapeDtypeStruct((N, D), jnp.float32),
            jax.ShapeDtypeStruct((N, D), jnp.float32),
            jax.ShapeDtypeStruct((N, D), jnp.float32),
        ],
    )(a, h, Wd, b12, Wb)


def _emm_body(q_ref, w_ref, b_ref, z_ref):
    z_ref[...] = jnp.dot(q_ref[...], w_ref[...],
                         preferred_element_type=jnp.float32) + b_ref[...]


def _emm(q, W2l, b22):
    return pl.pallas_call(
        _emm_body,
        grid=(QE // _EB,),
        in_specs=[
            pl.BlockSpec((_EB, D), lambda i: (i, 0)),
            pl.BlockSpec((D, D), lambda i: (0, 0)),
            pl.BlockSpec((1, D), lambda i: (0, 0)),
        ],
        out_specs=pl.BlockSpec((_EB, D), lambda i: (i, 0)),
        out_shape=jax.ShapeDtypeStruct((QE, D), jnp.float32),
    )(q, W2l, b22)


def _final_body(a_ref, h_ref, wo_ref, bo_ref, o_ref):
    hn = a_ref[...] + h_ref[...]
    o_ref[...] = jnp.dot(hn, wo_ref[...],
                         preferred_element_type=jnp.float32) + bo_ref[...]


def _final(a, h, Wo_pad, bo_pad):
    return pl.pallas_call(
        _final_body,
        grid=(N // _RB,),
        in_specs=[
            pl.BlockSpec((_RB, D), lambda i: (i, 0)),
            pl.BlockSpec((_RB, D), lambda i: (i, 0)),
            pl.BlockSpec((D, D), lambda i: (0, 0)),
            pl.BlockSpec((1, D), lambda i: (0, 0)),
        ],
        out_specs=pl.BlockSpec((_RB, D), lambda i: (i, 0)),
        out_shape=jax.ShapeDtypeStruct((N, D), jnp.float32),
    )(a, h, Wo_pad, bo_pad)


def kernel(x, edge_index, Wp, bp, W1, b1, W2, b2, Wo, bo):
    ei = edge_index.astype(jnp.int32)
    src, dst = ei[0], ei[1]

    bins, bsrc, counts = _bin_edges(dst, src)
    h = _proj(x, Wp, bp.reshape(1, D))

    agg = None
    for l in range(W1.shape[0]):
        Wd = W1[l, :D, :] - W1[l, D:, :]
        Wb = W1[l, D:, :]
        if agg is None:
            u, v = _uv0(h, Wd, b1[l].reshape(1, D), Wb)
        else:
            h, u, v = _uvres(agg, h, Wd, b1[l].reshape(1, D), Wb)
        q = _gather_q(u, v, bins, bsrc, counts)
        Z = _emm(q, W2[l], b2[l].reshape(1, D))
        aggp = _scatter_max(Z, bins, counts)
        aggp = aggp.reshape(8, NPAD, 16).transpose(1, 0, 2).reshape(NPAD, D)
        agg = aggp[:N]

    Wo_pad = jnp.pad(Wo, ((0, 0), (0, D - 1)))
    bo_pad = jnp.pad(bo.reshape(1, 1), ((0, 0), (0, D - 1)))
    out = _final(agg, h, Wo_pad, bo_pad)
    return out[:, 0]


# speedup vs baseline: 1.0140x; 1.0140x over previous
import functools

import jax
import jax.numpy as jnp
from jax import lax
from jax.experimental import pallas as pl
from jax.experimental.pallas import tpu as pltpu
from jax.experimental.pallas import tpu_sc as plsc

N = 10000
E = 320000
D = 128
NC = 2
NS = 16
NW = NC * NS
NPT = 320
NPAD = NW * NPT
EPT = E // NW
GC = 80
SCC = 128
DC = 2000
BKCAP = 10112
BLK = 128
SHIFT = 20
MASK = (1 << SHIFT) - 1
MAGIC = 52429
QE = 452608

_mesh = plsc.VectorSubcoreMesh(core_axis_name="c", subcore_axis_name="s")


def _wid():
    return lax.axis_index("s") * NC + lax.axis_index("c")


def _al8(v):
    return pl.multiple_of(v, 8)


@functools.partial(
    pl.kernel,
    out_type=(
        jax.ShapeDtypeStruct((NW * NW * BKCAP,), jnp.int32),
        jax.ShapeDtypeStruct((NW * NW * BKCAP,), jnp.int32),
        jax.ShapeDtypeStruct((NW * NW * 16,), jnp.int32),
    ),
    scratch_types=[
        pltpu.VMEM((DC,), jnp.int32),
        pltpu.VMEM((DC,), jnp.int32),
        pltpu.VMEM((NW * BLK,), jnp.int32),
        pltpu.VMEM((NW * BLK,), jnp.int32),
        pltpu.VMEM((NW * 16,), jnp.int32),
        pltpu.VMEM((NW * 16,), jnp.int32),
        pltpu.VMEM((NW * 16,), jnp.int32),
    ],
    mesh=_mesh,
)
def _bin_edges(dst_hbm, src_hbm, bins_hbm, bsrc_hbm, counts_hbm,
               dchunk, schunk, bblk, sblk, bcb, fbb, cnt):
    t = _wid()
    iota = lax.iota(jnp.int32, 16)
    zero = jnp.zeros((16,), jnp.int32)

    def zinit(k, _):
        bcb[pl.ds(k * 16, 16)] = zero
        fbb[pl.ds(k * 16, 16)] = zero
        return 0

    lax.fori_loop(0, NW, zinit, 0)

    def chunk_body(ch, _):
        ebase = t * EPT + ch * DC
        pltpu.sync_copy(dst_hbm.at[pl.ds(_al8(ebase), DC)], dchunk)
        pltpu.sync_copy(src_hbm.at[pl.ds(_al8(ebase), DC)], schunk)

        def vec_body(j, _):
            dvec = dchunk[pl.ds(j * 16, 16)]
            svec = schunk[pl.ds(j * 16, 16)]
            for i in range(16):
                d = dvec[i]
                s = svec[i]
                bkt = (d * MAGIC) >> 24
                val = ((d - bkt * NPT) << SHIFT) | (ebase + j * 16 + i)
                bcw = bcb[pl.ds(bkt * 16, 16)]
                bc = bcw[0]
                wpos = bkt * BLK + (bc & ~15)
                lane = iota == (bc & 15)
                w = bblk[pl.ds(wpos, 16)]
                bblk[pl.ds(wpos, 16)] = jnp.where(lane, val, w)
                w2 = sblk[pl.ds(wpos, 16)]
                sblk[pl.ds(wpos, 16)] = jnp.where(lane, s, w2)
                bcb[pl.ds(bkt * 16, 16)] = bcw + 1

                @pl.when(bc + 1 == BLK)
                def _():
                    fbw = fbb[pl.ds(bkt * 16, 16)]
                    dst_off = (t * NW + bkt) * BKCAP + fbw[0] * BLK
                    pltpu.sync_copy(
                        bblk.at[pl.ds(bkt * BLK, BLK)],
                        bins_hbm.at[pl.ds(_al8(dst_off), BLK)])
                    pltpu.sync_copy(
                        sblk.at[pl.ds(bkt * BLK, BLK)],
                        bsrc_hbm.at[pl.ds(_al8(dst_off), BLK)])
                    fbb[pl.ds(bkt * 16, 16)] = fbw + 1
                    bcb[pl.ds(bkt * 16, 16)] = zero
            return 0

        lax.fori_loop(0, DC // 16, vec_body, 0)
        return 0

    lax.fori_loop(0, EPT // DC, chunk_body, 0)

    for k in range(NW):
        bc = bcb[pl.ds(k * 16, 16)][0]
        fb = fbb[pl.ds(k * 16, 16)][0]
        dst_off = (t * NW + k) * BKCAP + fb * BLK
        pltpu.sync_copy(bblk.at[pl.ds(k * BLK, BLK)],
                        bins_hbm.at[pl.ds(_al8(dst_off), BLK)])
        pltpu.sync_copy(sblk.at[pl.ds(k * BLK, BLK)],
                        bsrc_hbm.at[pl.ds(_al8(dst_off), BLK)])
        cnt[pl.ds(k * 16, 16)] = jnp.full((16,), fb * BLK + bc, jnp.int32)
    pltpu.sync_copy(cnt, counts_hbm.at[pl.ds(_al8(t * NW * 16), NW * 16)])


@functools.partial(
    pl.kernel,
    out_type=jax.ShapeDtypeStruct((QE, D), jnp.float32),
    scratch_types=[
        pltpu.VMEM((NW * NW * 16,), jnp.int32),
        pltpu.VMEM((NPT, D), jnp.float32),
        pltpu.VMEM((SCC,), jnp.int32),
        pltpu.VMEM((SCC,), jnp.int32),
        pltpu.VMEM((SCC,), jnp.int32),
        pltpu.VMEM((SCC,), jnp.int32),
        pltpu.VMEM((SCC, D), jnp.float32),
        pltpu.VMEM((SCC, D), jnp.float32),
        pltpu.VMEM((SCC, D), jnp.float32),
        pltpu.VMEM((SCC, D), jnp.float32),
        pltpu.SemaphoreType.DMA,
        pltpu.SemaphoreType.DMA,
        pltpu.SemaphoreType.DMA,
        pltpu.SemaphoreType.DMA,
    ],
    mesh=_mesh,
)
def _gather_q(u_hbm, v_hbm, bins_hbm, bsrc_hbm, counts_hbm, q_hbm,
              cnts, ublk, pk0, pk1, ixs0, ixs1,
              vb0, vb1, qb0, qb1, sg0, sg1, so0, so1):
    k = _wid()
    iota = lax.iota(jnp.int32, 16)
    pltpu.sync_copy(counts_hbm, cnts)
    pltpu.sync_copy(u_hbm.at[pl.ds(_al8(k * NPT), NPT)], ublk)

    def accum(kp, acc):
        def accum_t(t, a):
            c = cnts[pl.ds((t * NW + kp) * 16, 16)][0]
            return a + ((c + 127) & ~127)
        return lax.fori_loop(0, NW, accum_t, acc)

    base = lax.fori_loop(0, k, accum, jnp.int32(0))

    def seg_body(t, qbase):
        ct = cnts[pl.ds((t * NW + k) * 16, 16)][0]
        nch = (ct + (SCC - 1)) >> 7
        sbase = (t * NW + k) * BKCAP

        def load_chunk(g, pk, ixs, vb, sg):
            pltpu.sync_copy(bins_hbm.at[pl.ds(_al8(sbase + g * SCC), SCC)], pk)
            pltpu.sync_copy(bsrc_hbm.at[pl.ds(_al8(sbase + g * SCC), SCC)], ixs)

            def idx_body(j, _):
                sl = pl.ds(j * 16, 16)
                valid = (g * SCC + j * 16 + iota) < ct
                ixs[sl] = jnp.where(valid, ixs[sl], 0)
                return 0

            lax.fori_loop(0, SCC // 16, idx_body, 0)
            pltpu.async_copy(v_hbm.at[ixs], vb, sg)

        @pl.when(nch > 0)
        def _():
            load_chunk(0, pk0, ixs0, vb0, sg0)

        def compute_chunk(g, pk, ixs, vb, sg, qb, so):
            pltpu.make_async_copy(v_hbm.at[ixs], vb, sg).wait()

            @pl.when(g >= 2)
            def _():
                pltpu.make_async_copy(qb, q_hbm.at[pl.ds(0, SCC)], so).wait()

            def grp(j, _):
                pv = pk[pl.ds(j * 16, 16)]
                valid = (g * SCC + j * 16 + iota) < ct
                dlv = jnp.where(valid, lax.shift_right_logical(pv, SHIFT), 0)
                for i in range(16):
                    u0 = dlv[i]
                    r = j * 16 + i
                    for kk in range(D // 16):
                        c = pl.ds(kk * 16, 16)
                        qb[r, c] = jnp.maximum(ublk[u0, c] + vb[r, c], 0.0)
                return 0

            lax.fori_loop(0, SCC // 16, grp, 0)
            pltpu.async_copy(qb, q_hbm.at[pl.ds(_al8(qbase + g * SCC), SCC)],
                             so)

        def halfstep(g, pk, ixs, vb, sg, qb, so, pkn, ixsn, vbn, sgn):
            @pl.when(g < nch)
            def _():
                @pl.when(g + 1 < nch)
                def _():
                    load_chunk(g + 1, pkn, ixsn, vbn, sgn)

                compute_chunk(g, pk, ixs, vb, sg, qb, so)

        def pair(pp, _):
            g = pp * 2
            halfstep(g, pk0, ixs0, vb0, sg0, qb0, so0, pk1, ixs1, vb1, sg1)
            halfstep(g + 1, pk1, ixs1, vb1, sg1, qb1, so1, pk0, ixs0, vb0, sg0)
            return 0

        lax.fori_loop(0, (nch + 1) >> 1, pair, 0)
        @pl.when(nch >= 2)
        def _():
            pltpu.make_async_copy(qb0, q_hbm.at[pl.ds(0, SCC)], so0).wait()
            pltpu.make_async_copy(qb1, q_hbm.at[pl.ds(0, SCC)], so1).wait()

        @pl.when(nch == 1)
        def _():
            pltpu.make_async_copy(qb0, q_hbm.at[pl.ds(0, SCC)], so0).wait()
        return qbase + ((ct + 127) & ~127)

    lax.fori_loop(0, NW, seg_body, base)


@functools.partial(
    pl.kernel,
    out_type=jax.ShapeDtypeStruct((8 * NPAD * 16,), jnp.float32),
    scratch_types=[
        pltpu.VMEM((NW * NW * 16,), jnp.int32),
        pltpu.VMEM((NW * 16,), jnp.int32),
        pltpu.VMEM((SCC,), jnp.int32),
        pltpu.VMEM((SCC,), jnp.int32),
        pltpu.VMEM((SCC, D), jnp.float32),
        pltpu.VMEM((SCC, D), jnp.float32),
    ] + [pltpu.VMEM(((NPT + 1) * 16,), jnp.float32) for _ in range(8)] + [
        pltpu.SemaphoreType.DMA,
        pltpu.SemaphoreType.DMA,
    ],
    mesh=_mesh,
)
def _scatter_max(z_hbm, bins_hbm, counts_hbm, agg_hbm,
                 cnts, qbv, pk0, pk1, zb0, zb1,
                 s0, s1, s2, s3, s4, s5, s6, s7, sz0, sz1):
    k = _wid()
    iota = lax.iota(jnp.int32, 16)
    zero = jnp.zeros((16,), jnp.float32)
    slabs = (s0, s1, s2, s3, s4, s5, s6, s7)

    def init_row(r, _):
        for kk in range(8):
            slabs[kk][pl.ds(r * 16, 16)] = zero
        return 0

    lax.fori_loop(0, NPT + 1, init_row, 0)

    pltpu.sync_copy(counts_hbm, cnts)

    def accum(kp, acc):
        def accum_t(t, a):
            c = cnts[pl.ds((t * NW + kp) * 16, 16)][0]
            return a + ((c + 127) & ~127)
        return lax.fori_loop(0, NW, accum_t, acc)

    base = lax.fori_loop(0, k, accum, jnp.int32(0))

    def setb(t, b):
        qbv[pl.ds(t * 16, 16)] = jnp.full((16,), b, jnp.int32)
        c = cnts[pl.ds((t * NW + k) * 16, 16)][0]
        return b + ((c + 127) & ~127)

    lax.fori_loop(0, NW, setb, base)

    def seg_body(t, _):
        ct = cnts[pl.ds((t * NW + k) * 16, 16)][0]
        qbase = qbv[pl.ds(t * 16, 16)][0]
        nch = (ct + (SCC - 1)) >> 7
        sbase = (t * NW + k) * BKCAP

        def load_chunk(g, pk, zb, sz):
            pltpu.sync_copy(bins_hbm.at[pl.ds(_al8(sbase + g * SCC), SCC)], pk)
            pltpu.async_copy(z_hbm.at[pl.ds(_al8(qbase + g * SCC), SCC)],
                             zb, sz)

        @pl.when(nch > 0)
        def _():
            load_chunk(0, pk0, zb0, sz0)

        def reduce_chunk(g, pk, zb, sz):
            pltpu.make_async_copy(z_hbm.at[pl.ds(0, SCC)], zb, sz).wait()

            def grp_body(j, _):
                p = pk[pl.ds(j * 16, 16)]
                valid = (g * SCC + j * 16 + iota) < ct
                dlv = jnp.where(valid, lax.shift_right_logical(p, SHIFT), NPT)
                for i in range(16):
                    r16 = dlv[i] * 16
                    e = j * 16 + i
                    for kk in range(8):
                        zv = zb[e, pl.ds(kk * 16, 16)]
                        av = slabs[kk][pl.ds(r16, 16)]
                        slabs[kk][pl.ds(r16, 16)] = jnp.maximum(av, zv)
                return 0

            lax.fori_loop(0, SCC // 16, grp_body, 0)

        def halfstep(g, pk, zb, sz, pkn, zbn, szn):
            @pl.when(g < nch)
            def _():
                @pl.when(g + 1 < nch)
                def _():
                    load_chunk(g + 1, pkn, zbn, szn)

                reduce_chunk(g, pk, zb, sz)

        def pair(pp, _):
            g = pp * 2
            halfstep(g, pk0, zb0, sz0, pk1, zb1, sz1)
            halfstep(g + 1, pk1, zb1, sz1, pk0, zb0, sz0)
            return 0

        lax.fori_loop(0, (nch + 1) >> 1, pair, 0)
        return 0

    lax.fori_loop(0, NW, seg_body, 0)
    for kk in range(8):
        pltpu.sync_copy(
            slabs[kk].at[pl.ds(0, NPT * 16)],
            agg_hbm.at[pl.ds(_al8(kk * NPAD * 16 + k * NPT * 16), NPT * 16)])


_RB = 1000
_EB = 2048


def _proj_body(x_ref, w_ref, b_ref, o_ref):
    o_ref[...] = jnp.dot(x_ref[...], w_ref[...],
                         preferred_element_type=jnp.float32) + b_ref[...]


def _proj(x, Wp, bp2):
    return pl.pallas_call(
        _proj_body,
        grid=(N // _RB,),
        in_specs=[
            pl.BlockSpec((_RB, D), lambda i: (i, 0)),
            pl.BlockSpec((D, D), lambda i: (0, 0)),
            pl.BlockSpec((1, D), lambda i: (0, 0)),
        ],
        out_specs=pl.BlockSpec((_RB, D), lambda i: (i, 0)),
        out_shape=jax.ShapeDtypeStruct((N, D), jnp.float32),
    )(x, Wp, bp2)


def _uv0_body(h_ref, wd_ref, b1_ref, wb_ref, u_ref, v_ref):
    h = h_ref[...]
    u_ref[...] = jnp.dot(h, wd_ref[...],
                         preferred_element_type=jnp.float32) + b1_ref[...]
    v_ref[...] = jnp.dot(h, wb_ref[...], preferred_element_type=jnp.float32)


def _uv0(h, Wd, b12, Wb):
    return pl.pallas_call(
        _uv0_body,
        grid=(N // _RB,),
        in_specs=[
            pl.BlockSpec((_RB, D), lambda i: (i, 0)),
            pl.BlockSpec((D, D), lambda i: (0, 0)),
            pl.BlockSpec((1, D), lambda i: (0, 0)),
            pl.BlockSpec((D, D), lambda i: (0, 0)),
        ],
        out_specs=[
            pl.BlockSpec((_RB, D), lambda i: (i, 0)),
            pl.BlockSpec((_RB, D), lambda i: (i, 0)),
        ],
        out_shape=[
            jax.ShapeDtypeStruct((N, D), jnp.float32),
            jax.ShapeDtypeStruct((N, D), jnp.float32),
        ],
    )(h, Wd, b12, Wb)


def _uvres_body(a_ref, h_ref, wd_ref, b1_ref, wb_ref, hn_ref, u_ref, v_ref):
    hn = a_ref[...] + h_ref[...]
    hn_ref[...] = hn
    u_ref[...] = jnp.dot(hn, wd_ref[...],
                         preferred_element_type=jnp.float32) + b1_ref[...]
    v_ref[...] = jnp.dot(hn, wb_ref[...], preferred_element_type=jnp.float32)


def _uvres(a, h, Wd, b12, Wb):
    return pl.pallas_call(
        _uvres_body,
        grid=(N // _RB,),
        in_specs=[
            pl.BlockSpec((_RB, D), lambda i: (i, 0)),
            pl.BlockSpec((_RB, D), lambda i: (i, 0)),
            pl.BlockSpec((D, D), lambda i: (0, 0)),
            pl.BlockSpec((1, D), lambda i: (0, 0)),
            pl.BlockSpec((D, D), lambda i: (0, 0)),
        ],
        out_specs=[
            pl.BlockSpec((_RB, D), lambda i: (i, 0)),
            pl.BlockSpec((_RB, D), lambda i: (i, 0)),
            pl.BlockSpec((_RB, D), lambda i: (i, 0)),
        ],
        out_shape=[
            jax.ShapeDtypeStruct((N, D), jnp.float32),
            jax.ShapeDtypeStruct((N, D), jnp.float32),
            jax.ShapeDtypeStruct((N, D), jnp.float32),
        ],
    )(a, h, Wd, b12, Wb)


def _emm_body(q_ref, w_ref, b_ref, z_ref):
    z_ref[...] = jnp.dot(q_ref[...], w_ref[...],
                         preferred_element_type=jnp.float32) + b_ref[...]


def _emm(q, W2l, b22):
    return pl.pallas_call(
        _emm_body,
        grid=(QE // _EB,),
        in_specs=[
            pl.BlockSpec((_EB, D), lambda i: (i, 0)),
            pl.BlockSpec((D, D), lambda i: (0, 0)),
            pl.BlockSpec((1, D), lambda i: (0, 0)),
        ],
        out_specs=pl.BlockSpec((_EB, D), lambda i: (i, 0)),
        out_shape=jax.ShapeDtypeStruct((QE, D), jnp.float32),
    )(q, W2l, b22)


def _final_body(a_ref, h_ref, wo_ref, bo_ref, o_ref):
    hn = a_ref[...] + h_ref[...]
    o_ref[...] = jnp.dot(hn, wo_ref[...],
                         preferred_element_type=jnp.float32) + bo_ref[...]


def _final(a, h, Wo_pad, bo_pad):
    return pl.pallas_call(
        _final_body,
        grid=(N // _RB,),
        in_specs=[
            pl.BlockSpec((_RB, D), lambda i: (i, 0)),
            pl.BlockSpec((_RB, D), lambda i: (i, 0)),
            pl.BlockSpec((D, D), lambda i: (0, 0)),
            pl.BlockSpec((1, D), lambda i: (0, 0)),
        ],
        out_specs=pl.BlockSpec((_RB, D), lambda i: (i, 0)),
        out_shape=jax.ShapeDtypeStruct((N, D), jnp.float32),
    )(a, h, Wo_pad, bo_pad)


def kernel(x, edge_index, Wp, bp, W1, b1, W2, b2, Wo, bo):
    ei = edge_index.astype(jnp.int32)
    src, dst = ei[0], ei[1]

    bins, bsrc, counts = _bin_edges(dst, src)
    h = _proj(x, Wp, bp.reshape(1, D))

    agg = None
    for l in range(W1.shape[0]):
        Wd = W1[l, :D, :] - W1[l, D:, :]
        Wb = W1[l, D:, :]
        if agg is None:
            u, v = _uv0(h, Wd, b1[l].reshape(1, D), Wb)
        else:
            h, u, v = _uvres(agg, h, Wd, b1[l].reshape(1, D), Wb)
        u_p = jnp.pad(u, ((0, NPAD - N), (0, 0)))
        q = _gather_q(u_p, v, bins, bsrc, counts)
        Z = _emm(q, W2[l], b2[l].reshape(1, D))
        aggp = _scatter_max(Z, bins, counts)
        aggp = aggp.reshape(8, NPAD, 16).transpose(1, 0, 2).reshape(NPAD, D)
        agg = aggp[:N]

    Wo_pad = jnp.pad(Wo, ((0, 0), (0, D - 1)))
    bo_pad = jnp.pad(bo.reshape(1, 1), ((0, 0), (0, D - 1)))
    out = _final(agg, h, Wo_pad, bo_pad)
    return out[:, 0]


# speedup vs baseline: 1.2640x; 1.2465x over previous
import functools

import jax
import jax.numpy as jnp
from jax import lax
from jax.experimental import pallas as pl
from jax.experimental.pallas import tpu as pltpu
from jax.experimental.pallas import tpu_sc as plsc

N = 10000
E = 320000
D = 128
NC = 2
NS = 16
NW = NC * NS
NPT = 320
NPAD = NW * NPT
EPT = E // NW
GC = 80
SCC = 128
DC = 2000
BKCAP = 10112
BLK = 128
SHIFT = 20
MASK = (1 << SHIFT) - 1
MAGIC = 52429

_mesh = plsc.VectorSubcoreMesh(core_axis_name="c", subcore_axis_name="s")


def _wid():
    return lax.axis_index("s") * NC + lax.axis_index("c")


def _al8(v):
    return pl.multiple_of(v, 8)


@functools.partial(
    pl.kernel,
    out_type=(
        jax.ShapeDtypeStruct((NW * NW * BKCAP,), jnp.int32),
        jax.ShapeDtypeStruct((NW * NW * 16,), jnp.int32),
    ),
    scratch_types=[
        pltpu.VMEM((DC,), jnp.int32),
        pltpu.VMEM((NW * BLK,), jnp.int32),
        pltpu.VMEM((NW * 16,), jnp.int32),
        pltpu.VMEM((NW * 16,), jnp.int32),
        pltpu.VMEM((NW * 16,), jnp.int32),
    ],
    mesh=_mesh,
)
def _bin_edges(dst_hbm, bins_hbm, counts_hbm, dchunk, bblk, bcb, fbb, cnt):
    t = _wid()
    iota = lax.iota(jnp.int32, 16)
    zero = jnp.zeros((16,), jnp.int32)

    def zinit(k, _):
        bcb[pl.ds(k * 16, 16)] = zero
        fbb[pl.ds(k * 16, 16)] = zero
        return 0

    lax.fori_loop(0, NW, zinit, 0)

    def chunk_body(ch, _):
        ebase = t * EPT + ch * DC
        pltpu.sync_copy(dst_hbm.at[pl.ds(_al8(ebase), DC)], dchunk)

        def vec_body(j, _):
            dvec = dchunk[pl.ds(j * 16, 16)]
            for i in range(16):
                d = dvec[i]
                bkt = (d * MAGIC) >> 24
                val = ((d - bkt * NPT) << SHIFT) | (ebase + j * 16 + i)
                bcw = bcb[pl.ds(bkt * 16, 16)]
                bc = bcw[0]
                wpos = bkt * BLK + (bc & ~15)
                w = bblk[pl.ds(wpos, 16)]
                bblk[pl.ds(wpos, 16)] = jnp.where(iota == (bc & 15), val, w)
                bcb[pl.ds(bkt * 16, 16)] = bcw + 1

                @pl.when(bc + 1 == BLK)
                def _():
                    fbw = fbb[pl.ds(bkt * 16, 16)]
                    dst_off = (t * NW + bkt) * BKCAP + fbw[0] * BLK
                    pltpu.sync_copy(
                        bblk.at[pl.ds(bkt * BLK, BLK)],
                        bins_hbm.at[pl.ds(_al8(dst_off), BLK)])
                    fbb[pl.ds(bkt * 16, 16)] = fbw + 1
                    bcb[pl.ds(bkt * 16, 16)] = zero
            return 0

        lax.fori_loop(0, DC // 16, vec_body, 0)
        return 0

    lax.fori_loop(0, EPT // DC, chunk_body, 0)

    for k in range(NW):
        bc = bcb[pl.ds(k * 16, 16)][0]
        fb = fbb[pl.ds(k * 16, 16)][0]
        dst_off = (t * NW + k) * BKCAP + fb * BLK
        pltpu.sync_copy(bblk.at[pl.ds(k * BLK, BLK)],
                        bins_hbm.at[pl.ds(_al8(dst_off), BLK)])
        cnt[pl.ds(k * 16, 16)] = jnp.full((16,), fb * BLK + bc, jnp.int32)
    pltpu.sync_copy(cnt, counts_hbm.at[pl.ds(_al8(t * NW * 16), NW * 16)])


@functools.partial(
    pl.kernel,
    out_type=jax.ShapeDtypeStruct((E, D), jnp.float32),
    scratch_types=[
        pltpu.VMEM((EPT,), jnp.int32),
        pltpu.VMEM((EPT,), jnp.int32),
        pltpu.VMEM((GC, D), jnp.float32),
        pltpu.VMEM((GC, D), jnp.float32),
        pltpu.VMEM((GC, D), jnp.float32),
        pltpu.VMEM((GC, D), jnp.float32),
        pltpu.VMEM((GC, D), jnp.float32),
        pltpu.VMEM((GC, D), jnp.float32),
        pltpu.SemaphoreType.DMA,
        pltpu.SemaphoreType.DMA,
        pltpu.SemaphoreType.DMA,
        pltpu.SemaphoreType.DMA,
    ],
    mesh=_mesh,
)
def _gather_q(u_hbm, v_hbm, dst_hbm, src_hbm, q_hbm,
              di, si, ub0, vb0, ub1, vb1, qb0, qb1, sg0, sg1, so0, so1):
    base = _wid() * EPT
    NCH = EPT // GC
    pltpu.sync_copy(dst_hbm.at[pl.ds(_al8(base), EPT)], di)
    pltpu.sync_copy(src_hbm.at[pl.ds(_al8(base), EPT)], si)

    def issue(g, ub, vb, sg):
        s = pl.ds(_al8(g * GC), GC)
        pltpu.async_copy(u_hbm.at[di.at[s]], ub, sg)
        pltpu.async_copy(v_hbm.at[si.at[s]], vb, sg)

    issue(0, ub0, vb0, sg0)

    def halfstep(g, ub, vb, sg, ubn, vbn, sgn, qb, so):
        @pl.when(g < NCH)
        def _():
            @pl.when(g + 1 < NCH)
            def _():
                issue(g + 1, ubn, vbn, sgn)

            pltpu.make_async_copy(u_hbm.at[di.at[pl.ds(0, GC)]], ub, sg).wait()
            pltpu.make_async_copy(v_hbm.at[si.at[pl.ds(0, GC)]], vb, sg).wait()

            @pl.when(g >= 2)
            def _():
                pltpu.make_async_copy(qb, q_hbm.at[pl.ds(0, GC)], so).wait()

            def row(r, _):
                for kk in range(D // 16):
                    c = pl.ds(kk * 16, 16)
                    qb[r, c] = jnp.maximum(ub[r, c] + vb[r, c], 0.0)
                return 0

            lax.fori_loop(0, GC, row, 0)
            pltpu.async_copy(qb, q_hbm.at[pl.ds(_al8(base + g * GC), GC)], so)

    def pair(p, _):
        g = p * 2
        halfstep(g, ub0, vb0, sg0, ub1, vb1, sg1, qb0, so0)
        halfstep(g + 1, ub1, vb1, sg1, ub0, vb0, sg0, qb1, so1)
        return 0

    lax.fori_loop(0, (NCH + 1) // 2, pair, 0)
    pltpu.make_async_copy(qb0, q_hbm.at[pl.ds(0, GC)], so0).wait()
    pltpu.make_async_copy(qb1, q_hbm.at[pl.ds(0, GC)], so1).wait()


@functools.partial(
    pl.kernel,
    out_type=jax.ShapeDtypeStruct((8 * NPAD * 16,), jnp.float32),
    scratch_types=[
        pltpu.VMEM((SCC,), jnp.int32),
        pltpu.VMEM((SCC,), jnp.int32),
        pltpu.VMEM((SCC,), jnp.int32),
        pltpu.VMEM((SCC,), jnp.int32),
        pltpu.VMEM((SCC, D), jnp.float32),
        pltpu.VMEM((SCC, D), jnp.float32),
    ] + [pltpu.VMEM(((NPT + 1) * 16,), jnp.float32) for _ in range(8)] + [
        pltpu.VMEM((16,), jnp.int32),
        pltpu.SemaphoreType.DMA,
        pltpu.SemaphoreType.DMA,
    ],
    mesh=_mesh,
)
def _scatter_max(z_hbm, bins_hbm, counts_hbm, agg_hbm,
                 pk0, pk1, eb0, eb1, zb0, zb1,
                 s0, s1, s2, s3, s4, s5, s6, s7, cb, sz0, sz1):
    k = _wid()
    iota = lax.iota(jnp.int32, 16)
    zero = jnp.zeros((16,), jnp.float32)
    slabs = (s0, s1, s2, s3, s4, s5, s6, s7)

    def init_row(r, _):
        for kk in range(8):
            slabs[kk][pl.ds(r * 16, 16)] = zero
        return 0

    lax.fori_loop(0, NPT + 1, init_row, 0)

    def seg_body(t, _):
        pltpu.sync_copy(counts_hbm.at[pl.ds(_al8((t * NW + k) * 16), 16)], cb)
        ct = cb[...][0]
        nch = (ct + (SCC - 1)) >> 7
        sbase = (t * NW + k) * BKCAP

        def load_chunk(g, pk, eb, zb, sz):
            pltpu.sync_copy(bins_hbm.at[pl.ds(_al8(sbase + g * SCC), SCC)], pk)

            def idx_body(j, _):
                p = pk[pl.ds(j * 16, 16)]
                valid = (g * SCC + j * 16 + iota) < ct
                eb[pl.ds(j * 16, 16)] = jnp.where(valid, p & MASK, 0)
                return 0

            lax.fori_loop(0, SCC // 16, idx_body, 0)
            pltpu.async_copy(z_hbm.at[eb], zb, sz)

        @pl.when(nch > 0)
        def _():
            load_chunk(0, pk0, eb0, zb0, sz0)

        def reduce_chunk(g, pk, eb, zb, sz):
            pltpu.make_async_copy(z_hbm.at[eb], zb, sz).wait()

            def grp_body(j, _):
                p = pk[pl.ds(j * 16, 16)]
                valid = (g * SCC + j * 16 + iota) < ct
                dlv = jnp.where(valid, lax.shift_right_logical(p, SHIFT), NPT)
                for i in range(16):
                    r16 = dlv[i] * 16
                    e = j * 16 + i
                    for kk in range(8):
                        zv = zb[e, pl.ds(kk * 16, 16)]
                        av = slabs[kk][pl.ds(r16, 16)]
                        slabs[kk][pl.ds(r16, 16)] = jnp.maximum(av, zv)
                return 0

            lax.fori_loop(0, SCC // 16, grp_body, 0)

        def halfstep(g, pk, eb, zb, sz, pkn, ebn, zbn, szn):
            @pl.when(g < nch)
            def _():
                @pl.when(g + 1 < nch)
                def _():
                    load_chunk(g + 1, pkn, ebn, zbn, szn)

                reduce_chunk(g, pk, eb, zb, sz)

        def pair(pp, _):
            g = pp * 2
            halfstep(g, pk0, eb0, zb0, sz0, pk1, eb1, zb1, sz1)
            halfstep(g + 1, pk1, eb1, zb1, sz1, pk0, eb0, zb0, sz0)
            return 0

        lax.fori_loop(0, (nch + 1) >> 1, pair, 0)
        return 0

    lax.fori_loop(0, NW, seg_body, 0)
    for kk in range(8):
        pltpu.sync_copy(
            slabs[kk].at[pl.ds(0, NPT * 16)],
            agg_hbm.at[pl.ds(_al8(kk * NPAD * 16 + k * NPT * 16), NPT * 16)])


_RB = 1000
_EB = 2000


def _proj_body(x_ref, w_ref, b_ref, o_ref):
    o_ref[...] = jnp.dot(x_ref[...], w_ref[...],
                         preferred_element_type=jnp.float32) + b_ref[...]


def _proj(x, Wp, bp2):
    return pl.pallas_call(
        _proj_body,
        grid=(N // _RB,),
        in_specs=[
            pl.BlockSpec((_RB, D), lambda i: (i, 0)),
            pl.BlockSpec((D, D), lambda i: (0, 0)),
            pl.BlockSpec((1, D), lambda i: (0, 0)),
        ],
        out_specs=pl.BlockSpec((_RB, D), lambda i: (i, 0)),
        out_shape=jax.ShapeDtypeStruct((N, D), jnp.float32),
    )(x, Wp, bp2)


def _uv0_body(h_ref, wd_ref, b1_ref, wb_ref, u_ref, v_ref):
    h = h_ref[...]
    u_ref[...] = jnp.dot(h, wd_ref[...],
                         preferred_element_type=jnp.float32) + b1_ref[...]
    v_ref[...] = jnp.dot(h, wb_ref[...], preferred_element_type=jnp.float32)


def _uv0(h, Wd, b12, Wb):
    return pl.pallas_call(
        _uv0_body,
        grid=(N // _RB,),
        in_specs=[
            pl.BlockSpec((_RB, D), lambda i: (i, 0)),
            pl.BlockSpec((D, D), lambda i: (0, 0)),
            pl.BlockSpec((1, D), lambda i: (0, 0)),
            pl.BlockSpec((D, D), lambda i: (0, 0)),
        ],
        out_specs=[
            pl.BlockSpec((_RB, D), lambda i: (i, 0)),
            pl.BlockSpec((_RB, D), lambda i: (i, 0)),
        ],
        out_shape=[
            jax.ShapeDtypeStruct((N, D), jnp.float32),
            jax.ShapeDtypeStruct((N, D), jnp.float32),
        ],
    )(h, Wd, b12, Wb)


def _uvres_body(a_ref, h_ref, wd_ref, b1_ref, wb_ref, hn_ref, u_ref, v_ref):
    hn = a_ref[...] + h_ref[...]
    hn_ref[...] = hn
    u_ref[...] = jnp.dot(hn, wd_ref[...],
                         preferred_element_type=jnp.float32) + b1_ref[...]
    v_ref[...] = jnp.dot(hn, wb_ref[...], preferred_element_type=jnp.float32)


def _uvres(a, h, Wd, b12, Wb):
    return pl.pallas_call(
        _uvres_body,
        grid=(N // _RB,),
        in_specs=[
            pl.BlockSpec((_RB, D), lambda i: (i, 0)),
            pl.BlockSpec((_RB, D), lambda i: (i, 0)),
            pl.BlockSpec((D, D), lambda i: (0, 0)),
            pl.BlockSpec((1, D), lambda i: (0, 0)),
            pl.BlockSpec((D, D), lambda i: (0, 0)),
        ],
        out_specs=[
            pl.BlockSpec((_RB, D), lambda i: (i, 0)),
            pl.BlockSpec((_RB, D), lambda i: (i, 0)),
            pl.BlockSpec((_RB, D), lambda i: (i, 0)),
        ],
        out_shape=[
            jax.ShapeDtypeStruct((N, D), jnp.float32),
            jax.ShapeDtypeStruct((N, D), jnp.float32),
            jax.ShapeDtypeStruct((N, D), jnp.float32),
        ],
    )(a, h, Wd, b12, Wb)


def _emm_body(q_ref, w_ref, b_ref, z_ref):
    z_ref[...] = jnp.dot(q_ref[...], w_ref[...],
                         preferred_element_type=jnp.float32) + b_ref[...]


def _emm(q, W2l, b22):
    return pl.pallas_call(
        _emm_body,
        grid=(E // _EB,),
        in_specs=[
            pl.BlockSpec((_EB, D), lambda i: (i, 0)),
            pl.BlockSpec((D, D), lambda i: (0, 0)),
            pl.BlockSpec((1, D), lambda i: (0, 0)),
        ],
        out_specs=pl.BlockSpec((_EB, D), lambda i: (i, 0)),
        out_shape=jax.ShapeDtypeStruct((E, D), jnp.float32),
    )(q, W2l, b22)


def _final_body(a_ref, h_ref, wo_ref, bo_ref, o_ref):
    hn = a_ref[...] + h_ref[...]
    o_ref[...] = jnp.dot(hn, wo_ref[...],
                         preferred_element_type=jnp.float32) + bo_ref[...]


def _final(a, h, Wo_pad, bo_pad):
    return pl.pallas_call(
        _final_body,
        grid=(N // _RB,),
        in_specs=[
            pl.BlockSpec((_RB, D), lambda i: (i, 0)),
            pl.BlockSpec((_RB, D), lambda i: (i, 0)),
            pl.BlockSpec((D, D), lambda i: (0, 0)),
            pl.BlockSpec((1, D), lambda i: (0, 0)),
        ],
        out_specs=pl.BlockSpec((_RB, D), lambda i: (i, 0)),
        out_shape=jax.ShapeDtypeStruct((N, D), jnp.float32),
    )(a, h, Wo_pad, bo_pad)


def kernel(x, edge_index, Wp, bp, W1, b1, W2, b2, Wo, bo):
    ei = edge_index.astype(jnp.int32)
    src, dst = ei[0], ei[1]

    lists, counts = _bin_edges(dst)
    h = _proj(x, Wp, bp.reshape(1, D))

    agg = None
    for l in range(W1.shape[0]):
        Wd = W1[l, :D, :] - W1[l, D:, :]
        Wb = W1[l, D:, :]
        if agg is None:
            u, v = _uv0(h, Wd, b1[l].reshape(1, D), Wb)
        else:
            h, u, v = _uvres(agg, h, Wd, b1[l].reshape(1, D), Wb)
        q = _gather_q(u, v, dst, src)
        Z = _emm(q, W2[l], b2[l].reshape(1, D))
        aggp = _scatter_max(Z, lists, counts)
        aggp = aggp.reshape(8, NPAD, 16).transpose(1, 0, 2).reshape(NPAD, D)
        agg = aggp[:N]

    Wo_pad = jnp.pad(Wo, ((0, 0), (0, D - 1)))
    bo_pad = jnp.pad(bo.reshape(1, 1), ((0, 0), (0, D - 1)))
    out = _final(agg, h, Wo_pad, bo_pad)
    return out[:, 0]


# speedup vs baseline: 1.2656x; 1.0013x over previous
import functools

import jax
import jax.numpy as jnp
from jax import lax
from jax.experimental import pallas as pl
from jax.experimental.pallas import tpu as pltpu
from jax.experimental.pallas import tpu_sc as plsc

N = 10000
E = 320000
D = 128
NC = 2
NS = 16
NW = NC * NS
NPT = 320
NPAD = NW * NPT
EPT = E // NW
GC = 80
SCC = 128
DC = 2000
BKCAP = 10112
BLK = 128
SHIFT = 20
MASK = (1 << SHIFT) - 1
MAGIC = 52429

_mesh = plsc.VectorSubcoreMesh(core_axis_name="c", subcore_axis_name="s")


def _wid():
    return lax.axis_index("s") * NC + lax.axis_index("c")


def _al8(v):
    return pl.multiple_of(v, 8)


@functools.partial(
    pl.kernel,
    out_type=(
        jax.ShapeDtypeStruct((NW * NW * BKCAP,), jnp.int32),
        jax.ShapeDtypeStruct((NW * NW * 16,), jnp.int32),
    ),
    scratch_types=[
        pltpu.VMEM((DC,), jnp.int32),
        pltpu.VMEM((NW * BLK,), jnp.int32),
        pltpu.VMEM((NW * 16,), jnp.int32),
        pltpu.VMEM((NW * 16,), jnp.int32),
        pltpu.VMEM((NW * 16,), jnp.int32),
    ],
    mesh=_mesh,
)
def _bin_edges(dst_hbm, bins_hbm, counts_hbm, dchunk, bblk, bcb, fbb, cnt):
    t = _wid()
    iota = lax.iota(jnp.int32, 16)
    zero = jnp.zeros((16,), jnp.int32)

    def zinit(k, _):
        bcb[pl.ds(k * 16, 16)] = zero
        fbb[pl.ds(k * 16, 16)] = zero
        return 0

    lax.fori_loop(0, NW, zinit, 0)

    def chunk_body(ch, _):
        ebase = t * EPT + ch * DC
        pltpu.sync_copy(dst_hbm.at[pl.ds(_al8(ebase), DC)], dchunk)

        def vec_body(j, _):
            dvec = dchunk[pl.ds(j * 16, 16)]
            for i in range(16):
                d = dvec[i]
                bkt = (d * MAGIC) >> 24
                val = ((d - bkt * NPT) << SHIFT) | (ebase + j * 16 + i)
                bcw = bcb[pl.ds(bkt * 16, 16)]
                bc = bcw[0]
                wpos = bkt * BLK + (bc & ~15)
                w = bblk[pl.ds(wpos, 16)]
                bblk[pl.ds(wpos, 16)] = jnp.where(iota == (bc & 15), val, w)
                bcb[pl.ds(bkt * 16, 16)] = bcw + 1

                @pl.when(bc + 1 == BLK)
                def _():
                    fbw = fbb[pl.ds(bkt * 16, 16)]
                    dst_off = (t * NW + bkt) * BKCAP + fbw[0] * BLK
                    pltpu.sync_copy(
                        bblk.at[pl.ds(bkt * BLK, BLK)],
                        bins_hbm.at[pl.ds(_al8(dst_off), BLK)])
                    fbb[pl.ds(bkt * 16, 16)] = fbw + 1
                    bcb[pl.ds(bkt * 16, 16)] = zero
            return 0

        lax.fori_loop(0, DC // 16, vec_body, 0)
        return 0

    lax.fori_loop(0, EPT // DC, chunk_body, 0)

    for k in range(NW):
        bc = bcb[pl.ds(k * 16, 16)][0]
        fb = fbb[pl.ds(k * 16, 16)][0]
        dst_off = (t * NW + k) * BKCAP + fb * BLK
        pltpu.sync_copy(bblk.at[pl.ds(k * BLK, BLK)],
                        bins_hbm.at[pl.ds(_al8(dst_off), BLK)])
        cnt[pl.ds(k * 16, 16)] = jnp.full((16,), fb * BLK + bc, jnp.int32)
    pltpu.sync_copy(cnt, counts_hbm.at[pl.ds(_al8(t * NW * 16), NW * 16)])


@functools.partial(
    pl.kernel,
    out_type=jax.ShapeDtypeStruct((E, D), jnp.float32),
    scratch_types=[
        pltpu.VMEM((EPT,), jnp.int32),
        pltpu.VMEM((EPT,), jnp.int32),
        pltpu.VMEM((GC, D), jnp.float32),
        pltpu.VMEM((GC, D), jnp.float32),
        pltpu.VMEM((GC, D), jnp.float32),
        pltpu.VMEM((GC, D), jnp.float32),
        pltpu.VMEM((GC, D), jnp.float32),
        pltpu.VMEM((GC, D), jnp.float32),
        pltpu.SemaphoreType.DMA,
        pltpu.SemaphoreType.DMA,
        pltpu.SemaphoreType.DMA,
        pltpu.SemaphoreType.DMA,
    ],
    mesh=_mesh,
)
def _gather_q(u_hbm, v_hbm, dst_hbm, src_hbm, q_hbm,
              di, si, ub0, vb0, ub1, vb1, qb0, qb1, sg0, sg1, so0, so1):
    base = _wid() * EPT
    NCH = EPT // GC
    pltpu.sync_copy(dst_hbm.at[pl.ds(_al8(base), EPT)], di)
    pltpu.sync_copy(src_hbm.at[pl.ds(_al8(base), EPT)], si)

    def issue(g, ub, vb, sg):
        s = pl.ds(_al8(g * GC), GC)
        pltpu.async_copy(u_hbm.at[di.at[s]], ub, sg)
        pltpu.async_copy(v_hbm.at[si.at[s]], vb, sg)

    issue(0, ub0, vb0, sg0)

    def halfstep(g, ub, vb, sg, ubn, vbn, sgn, qb, so):
        @pl.when(g < NCH)
        def _():
            @pl.when(g + 1 < NCH)
            def _():
                issue(g + 1, ubn, vbn, sgn)

            pltpu.make_async_copy(u_hbm.at[di.at[pl.ds(0, GC)]], ub, sg).wait()
            pltpu.make_async_copy(v_hbm.at[si.at[pl.ds(0, GC)]], vb, sg).wait()

            @pl.when(g >= 2)
            def _():
                pltpu.make_async_copy(qb, q_hbm.at[pl.ds(0, GC)], so).wait()

            def row(r, _):
                for kk in range(D // 16):
                    c = pl.ds(kk * 16, 16)
                    qb[r, c] = jnp.maximum(ub[r, c] + vb[r, c], 0.0)
                return 0

            lax.fori_loop(0, GC, row, 0)
            pltpu.async_copy(qb, q_hbm.at[pl.ds(_al8(base + g * GC), GC)], so)

    def pair(p, _):
        g = p * 2
        halfstep(g, ub0, vb0, sg0, ub1, vb1, sg1, qb0, so0)
        halfstep(g + 1, ub1, vb1, sg1, ub0, vb0, sg0, qb1, so1)
        return 0

    lax.fori_loop(0, (NCH + 1) // 2, pair, 0)
    pltpu.make_async_copy(qb0, q_hbm.at[pl.ds(0, GC)], so0).wait()
    pltpu.make_async_copy(qb1, q_hbm.at[pl.ds(0, GC)], so1).wait()


@functools.partial(
    pl.kernel,
    out_type=jax.ShapeDtypeStruct((8 * NPAD * 16,), jnp.float32),
    scratch_types=[
        pltpu.VMEM((SCC,), jnp.int32),
        pltpu.VMEM((SCC,), jnp.int32),
        pltpu.VMEM((SCC,), jnp.int32),
        pltpu.VMEM((SCC,), jnp.int32),
        pltpu.VMEM((SCC, D), jnp.float32),
        pltpu.VMEM((SCC, D), jnp.float32),
    ] + [pltpu.VMEM(((NPT + 1) * 16,), jnp.float32) for _ in range(8)] + [
        pltpu.VMEM((16,), jnp.int32),
        pltpu.SemaphoreType.DMA,
        pltpu.SemaphoreType.DMA,
    ],
    mesh=_mesh,
)
def _scatter_max(z_hbm, bins_hbm, counts_hbm, agg_hbm,
                 pk0, pk1, eb0, eb1, zb0, zb1,
                 s0, s1, s2, s3, s4, s5, s6, s7, cb, sz0, sz1):
    k = _wid()
    iota = lax.iota(jnp.int32, 16)
    zero = jnp.zeros((16,), jnp.float32)
    slabs = (s0, s1, s2, s3, s4, s5, s6, s7)

    def init_row(r, _):
        for kk in range(8):
            slabs[kk][pl.ds(r * 16, 16)] = zero
        return 0

    lax.fori_loop(0, NPT + 1, init_row, 0)

    def seg_body(t, _):
        pltpu.sync_copy(counts_hbm.at[pl.ds(_al8((t * NW + k) * 16), 16)], cb)
        ct = cb[...][0]
        nch = (ct + (SCC - 1)) >> 7
        sbase = (t * NW + k) * BKCAP

        def load_chunk(g, pk, eb, zb, sz):
            pltpu.sync_copy(bins_hbm.at[pl.ds(_al8(sbase + g * SCC), SCC)], pk)

            def idx_body(j, _):
                p = pk[pl.ds(j * 16, 16)]
                valid = (g * SCC + j * 16 + iota) < ct
                eb[pl.ds(j * 16, 16)] = jnp.where(valid, p & MASK, 0)
                return 0

            lax.fori_loop(0, SCC // 16, idx_body, 0)
            for q4 in range(4):
                r4 = pl.ds(q4 * 32, 32)
                pltpu.async_copy(z_hbm.at[eb.at[r4]], zb.at[r4], sz)

        @pl.when(nch > 0)
        def _():
            load_chunk(0, pk0, eb0, zb0, sz0)

        def reduce_chunk(g, pk, eb, zb, sz):
            pltpu.make_async_copy(z_hbm.at[eb], zb, sz).wait()

            def grp_body(j, _):
                p = pk[pl.ds(j * 16, 16)]
                valid = (g * SCC + j * 16 + iota) < ct
                dlv = jnp.where(valid, lax.shift_right_logical(p, SHIFT), NPT)
                for i in range(16):
                    r16 = dlv[i] * 16
                    e = j * 16 + i
                    for kk in range(8):
                        zv = zb[e, pl.ds(kk * 16, 16)]
                        av = slabs[kk][pl.ds(r16, 16)]
                        slabs[kk][pl.ds(r16, 16)] = jnp.maximum(av, zv)
                return 0

            lax.fori_loop(0, SCC // 16, grp_body, 0)

        def halfstep(g, pk, eb, zb, sz, pkn, ebn, zbn, szn):
            @pl.when(g < nch)
            def _():
                @pl.when(g + 1 < nch)
                def _():
                    load_chunk(g + 1, pkn, ebn, zbn, szn)

                reduce_chunk(g, pk, eb, zb, sz)

        def pair(pp, _):
            g = pp * 2
            halfstep(g, pk0, eb0, zb0, sz0, pk1, eb1, zb1, sz1)
            halfstep(g + 1, pk1, eb1, zb1, sz1, pk0, eb0, zb0, sz0)
            return 0

        lax.fori_loop(0, (nch + 1) >> 1, pair, 0)
        return 0

    lax.fori_loop(0, NW, seg_body, 0)
    for kk in range(8):
        pltpu.sync_copy(
            slabs[kk].at[pl.ds(0, NPT * 16)],
            agg_hbm.at[pl.ds(_al8(kk * NPAD * 16 + k * NPT * 16), NPT * 16)])


_RB = 1000
_EB = 2000


def _proj_body(x_ref, w_ref, b_ref, o_ref):
    o_ref[...] = jnp.dot(x_ref[...], w_ref[...],
                         preferred_element_type=jnp.float32) + b_ref[...]


def _proj(x, Wp, bp2):
    return pl.pallas_call(
        _proj_body,
        grid=(N // _RB,),
        in_specs=[
            pl.BlockSpec((_RB, D), lambda i: (i, 0)),
            pl.BlockSpec((D, D), lambda i: (0, 0)),
            pl.BlockSpec((1, D), lambda i: (0, 0)),
        ],
        out_specs=pl.BlockSpec((_RB, D), lambda i: (i, 0)),
        out_shape=jax.ShapeDtypeStruct((N, D), jnp.float32),
    )(x, Wp, bp2)


def _uv0_body(h_ref, wd_ref, b1_ref, wb_ref, u_ref, v_ref):
    h = h_ref[...]
    u_ref[...] = jnp.dot(h, wd_ref[...],
                         preferred_element_type=jnp.float32) + b1_ref[...]
    v_ref[...] = jnp.dot(h, wb_ref[...], preferred_element_type=jnp.float32)


def _uv0(h, Wd, b12, Wb):
    return pl.pallas_call(
        _uv0_body,
        grid=(N // _RB,),
        in_specs=[
            pl.BlockSpec((_RB, D), lambda i: (i, 0)),
            pl.BlockSpec((D, D), lambda i: (0, 0)),
            pl.BlockSpec((1, D), lambda i: (0, 0)),
            pl.BlockSpec((D, D), lambda i: (0, 0)),
        ],
        out_specs=[
            pl.BlockSpec((_RB, D), lambda i: (i, 0)),
            pl.BlockSpec((_RB, D), lambda i: (i, 0)),
        ],
        out_shape=[
            jax.ShapeDtypeStruct((N, D), jnp.float32),
            jax.ShapeDtypeStruct((N, D), jnp.float32),
        ],
    )(h, Wd, b12, Wb)


def _uvres_body(a_ref, h_ref, wd_ref, b1_ref, wb_ref, hn_ref, u_ref, v_ref):
    hn = a_ref[...] + h_ref[...]
    hn_ref[...] = hn
    u_ref[...] = jnp.dot(hn, wd_ref[...],
                         preferred_element_type=jnp.float32) + b1_ref[...]
    v_ref[...] = jnp.dot(hn, wb_ref[...], preferred_element_type=jnp.float32)


def _uvres(a, h, Wd, b12, Wb):
    return pl.pallas_call(
        _uvres_body,
        grid=(N // _RB,),
        in_specs=[
            pl.BlockSpec((_RB, D), lambda i: (i, 0)),
            pl.BlockSpec((_RB, D), lambda i: (i, 0)),
            pl.BlockSpec((D, D), lambda i: (0, 0)),
            pl.BlockSpec((1, D), lambda i: (0, 0)),
            pl.BlockSpec((D, D), lambda i: (0, 0)),
        ],
        out_specs=[
            pl.BlockSpec((_RB, D), lambda i: (i, 0)),
            pl.BlockSpec((_RB, D), lambda i: (i, 0)),
            pl.BlockSpec((_RB, D), lambda i: (i, 0)),
        ],
        out_shape=[
            jax.ShapeDtypeStruct((N, D), jnp.float32),
            jax.ShapeDtypeStruct((N, D), jnp.float32),
            jax.ShapeDtypeStruct((N, D), jnp.float32),
        ],
    )(a, h, Wd, b12, Wb)


def _emm_body(q_ref, w_ref, b_ref, z_ref):
    z_ref[...] = jnp.dot(q_ref[...], w_ref[...],
                         preferred_element_type=jnp.float32) + b_ref[...]


def _emm(q, W2l, b22):
    return pl.pallas_call(
        _emm_body,
        grid=(E // _EB,),
        in_specs=[
            pl.BlockSpec((_EB, D), lambda i: (i, 0)),
            pl.BlockSpec((D, D), lambda i: (0, 0)),
            pl.BlockSpec((1, D), lambda i: (0, 0)),
        ],
        out_specs=pl.BlockSpec((_EB, D), lambda i: (i, 0)),
        out_shape=jax.ShapeDtypeStruct((E, D), jnp.float32),
    )(q, W2l, b22)


def _final_body(a_ref, h_ref, wo_ref, bo_ref, o_ref):
    hn = a_ref[...] + h_ref[...]
    o_ref[...] = jnp.dot(hn, wo_ref[...],
                         preferred_element_type=jnp.float32) + bo_ref[...]


def _final(a, h, Wo_pad, bo_pad):
    return pl.pallas_call(
        _final_body,
        grid=(N // _RB,),
        in_specs=[
            pl.BlockSpec((_RB, D), lambda i: (i, 0)),
            pl.BlockSpec((_RB, D), lambda i: (i, 0)),
            pl.BlockSpec((D, D), lambda i: (0, 0)),
            pl.BlockSpec((1, D), lambda i: (0, 0)),
        ],
        out_specs=pl.BlockSpec((_RB, D), lambda i: (i, 0)),
        out_shape=jax.ShapeDtypeStruct((N, D), jnp.float32),
    )(a, h, Wo_pad, bo_pad)


def kernel(x, edge_index, Wp, bp, W1, b1, W2, b2, Wo, bo):
    ei = edge_index.astype(jnp.int32)
    src, dst = ei[0], ei[1]

    lists, counts = _bin_edges(dst)
    h = _proj(x, Wp, bp.reshape(1, D))

    agg = None
    for l in range(W1.shape[0]):
        Wd = W1[l, :D, :] - W1[l, D:, :]
        Wb = W1[l, D:, :]
        if agg is None:
            u, v = _uv0(h, Wd, b1[l].reshape(1, D), Wb)
        else:
            h, u, v = _uvres(agg, h, Wd, b1[l].reshape(1, D), Wb)
        q = _gather_q(u, v, dst, src)
        Z = _emm(q, W2[l], b2[l].reshape(1, D))
        aggp = _scatter_max(Z, lists, counts)
        aggp = aggp.reshape(8, NPAD, 16).transpose(1, 0, 2).reshape(NPAD, D)
        agg = aggp[:N]

    Wo_pad = jnp.pad(Wo, ((0, 0), (0, D - 1)))
    bo_pad = jnp.pad(bo.reshape(1, 1), ((0, 0), (0, D - 1)))
    out = _final(agg, h, Wo_pad, bo_pad)
    return out[:, 0]
